# baseline (device time: 42756 ns/iter reference)
import jax
import jax.numpy as jnp
from jax import lax
from jax.experimental import pallas as pl
from jax.experimental.pallas import tpu as pltpu

N_DEV = 4
SUB = 4
LANE = 128
MAX_BIT = 10


def kernel(x, dest):
    n_rows, n_cols = x.shape
    assert n_cols == SUB * LANE
    dest = dest.astype(jnp.int32)

    onehot = (dest[:, None] == jnp.arange(N_DEV, dtype=jnp.int32)[None, :])
    onehot = onehot.astype(jnp.int32)
    cnt = onehot.sum(axis=0, dtype=jnp.int32)
    rank = jnp.take_along_axis(
        jnp.cumsum(onehot, axis=0) - 1, dest[:, None], axis=1
    )[:, 0]
    loff = jnp.concatenate(
        [jnp.zeros((1,), jnp.int32), jnp.cumsum(cnt)[:-1]]
    )
    pos = loff[dest] + rank
    order = jnp.zeros((n_rows,), jnp.int32).at[pos].set(
        jnp.arange(n_rows, dtype=jnp.int32)
    )
    xs = x.astype(jnp.bfloat16)[order].reshape(n_rows, SUB, LANE)
    cnt_pad = jnp.zeros((1, LANE), jnp.int32).at[0, :N_DEV].set(cnt)

    def body(xs_ref, cnt_ref, out_ref,
             cnt_all_ref, cnt_sm_ref, myc_ref, loff_ref, base_ref, incc_ref,
             cnt_send_sem, cnt_recv_sem, send_sem, recv_sem, local_sem):
        my = lax.axis_index("i")

        bar = pltpu.get_barrier_semaphore()
        for k in range(1, N_DEV):
            pl.semaphore_signal(
                bar, inc=1, device_id=((my + k) % N_DEV,),
                device_id_type=pl.DeviceIdType.MESH,
            )
        pl.semaphore_wait(bar, N_DEV - 1)

        cnt_all_ref[pl.ds(my, 1)] = cnt_ref[:][None]
        for k in range(1, N_DEV):
            pltpu.make_async_remote_copy(
                src_ref=cnt_all_ref.at[pl.ds(my, 1)],
                dst_ref=cnt_all_ref.at[pl.ds(my, 1)],
                send_sem=cnt_send_sem, recv_sem=cnt_recv_sem,
                device_id=((my + k) % N_DEV,),
                device_id_type=pl.DeviceIdType.MESH,
            ).start()
        cnt_dummy = pltpu.make_async_remote_copy(
            src_ref=cnt_all_ref.at[pl.ds(0, 1)],
            dst_ref=cnt_all_ref.at[pl.ds(0, 1)],
            send_sem=cnt_send_sem, recv_sem=cnt_recv_sem,
            device_id=(my,), device_id_type=pl.DeviceIdType.MESH,
        )
        for _ in range(N_DEV - 1):
            cnt_dummy.wait_send()
        for _ in range(N_DEV - 1):
            cnt_dummy.wait_recv()

        cp = pltpu.make_async_copy(cnt_all_ref, cnt_sm_ref, local_sem)
        cp.start()
        cp.wait()

        for d in range(N_DEV):
            m = jnp.int32(0)
            lo = jnp.int32(0)
            b = jnp.int32(0)
            for s in range(N_DEV):
                c_sd = cnt_sm_ref[s, 0, d]
                m = m + jnp.where(my == s, c_sd, 0)
                b = b + jnp.where(my > s, c_sd, 0)
            for dp in range(d):
                for s in range(N_DEV):
                    lo = lo + jnp.where(my == s, cnt_sm_ref[s, 0, dp], 0)
            myc_ref[d] = m
            loff_ref[d] = lo
            base_ref[d] = b
        for s in range(N_DEV):
            ic = jnp.int32(0)
            for d in range(N_DEV):
                ic = ic + jnp.where(my == d, cnt_sm_ref[s, 0, d], 0)
            incc_ref[s] = ic

        def chunk_off(c, sz):
            return c & jnp.int32(~(2 * sz - 1))

        c_loc = jnp.int32(0)
        for d in range(N_DEV):
            c_loc = c_loc + jnp.where(my == d, cnt_sm_ref[d, 0, d], 0)
        lo_my = loff_ref[my]
        b_my = base_ref[my]
        for bit in range(MAX_BIT, -1, -1):
            sz = 1 << bit

            @pl.when((c_loc & sz) != 0)
            def _(sz=sz):
                off = chunk_off(c_loc, sz)
                pltpu.make_async_copy(
                    xs_ref.at[pl.ds(lo_my + off, sz)],
                    out_ref.at[pl.ds(b_my + off, sz)],
                    local_sem,
                ).start()

        for k in range(1, N_DEV):
            dd = (my + k) % N_DEV
            c = myc_ref[dd]
            s0 = loff_ref[dd]
            r0 = base_ref[dd]
            for bit in range(MAX_BIT, -1, -1):
                sz = 1 << bit

                @pl.when((c & sz) != 0)
                def _(sz=sz, c=c, s0=s0, r0=r0, dd=dd):
                    off = chunk_off(c, sz)
                    pltpu.make_async_remote_copy(
                        src_ref=xs_ref.at[pl.ds(s0 + off, sz)],
                        dst_ref=out_ref.at[pl.ds(r0 + off, sz)],
                        send_sem=send_sem, recv_sem=recv_sem,
                        device_id=(dd,),
                        device_id_type=pl.DeviceIdType.MESH,
                    ).start()

        def dummy_for(sz):
            return pltpu.make_async_remote_copy(
                src_ref=xs_ref.at[pl.ds(0, sz)],
                dst_ref=out_ref.at[pl.ds(0, sz)],
                send_sem=send_sem, recv_sem=recv_sem,
                device_id=(my,), device_id_type=pl.DeviceIdType.MESH,
            )

        for bit in range(MAX_BIT, -1, -1):
            sz = 1 << bit

            @pl.when((c_loc & sz) != 0)
            def _(sz=sz):
                pltpu.make_async_copy(
                    xs_ref.at[pl.ds(0, sz)],
                    out_ref.at[pl.ds(0, sz)],
                    local_sem,
                ).wait()

        for k in range(1, N_DEV):
            c = myc_ref[(my + k) % N_DEV]
            for bit in range(MAX_BIT, -1, -1):
                sz = 1 << bit

                @pl.when((c & sz) != 0)
                def _(sz=sz):
                    dummy_for(sz).wait_send()

        for k in range(1, N_DEV):
            c = incc_ref[(my + k) % N_DEV]
            for bit in range(MAX_BIT, -1, -1):
                sz = 1 << bit

                @pl.when((c & sz) != 0)
                def _(sz=sz):
                    dummy_for(sz).wait_recv()

    out3 = pl.pallas_call(
        body,
        out_shape=jax.ShapeDtypeStruct((n_rows, SUB, LANE), jnp.bfloat16),
        in_specs=[
            pl.BlockSpec(memory_space=pltpu.VMEM),
            pl.BlockSpec(memory_space=pltpu.VMEM),
        ],
        out_specs=pl.BlockSpec(memory_space=pltpu.VMEM),
        scratch_shapes=[
            pltpu.VMEM((N_DEV, 1, LANE), jnp.int32),
            pltpu.SMEM((N_DEV, 1, LANE), jnp.int32),
            pltpu.SMEM((N_DEV,), jnp.int32),
            pltpu.SMEM((N_DEV,), jnp.int32),
            pltpu.SMEM((N_DEV,), jnp.int32),
            pltpu.SMEM((N_DEV,), jnp.int32),
            pltpu.SemaphoreType.DMA,
            pltpu.SemaphoreType.DMA,
            pltpu.SemaphoreType.DMA,
            pltpu.SemaphoreType.DMA,
            pltpu.SemaphoreType.DMA,
        ],
        compiler_params=pltpu.CompilerParams(collective_id=0),
    )(xs, cnt_pad)
    return out3.reshape(n_rows, n_cols)


# device time: 25489 ns/iter; 1.6774x vs baseline; 1.6774x over previous
import jax
import jax.numpy as jnp
from jax import lax
from jax.experimental import pallas as pl
from jax.experimental.pallas import tpu as pltpu

N_DEV = 4
SUB = 4
LANE = 128
MAX_BIT = 10


def kernel(x, dest):
    n_rows, n_cols = x.shape
    assert n_cols == SUB * LANE
    dest = dest.astype(jnp.int32)

    key = dest * jnp.int32(2 * n_rows) + jnp.arange(n_rows, dtype=jnp.int32)
    order = jnp.sort(key) & jnp.int32(2 * n_rows - 1)
    xs = x[order].astype(jnp.bfloat16).reshape(n_rows, SUB, LANE)
    dest2d = dest[:, None]

    def body(xs_ref, dest_ref, out_ref,
             cnt_all_ref, cnt_sm_ref, myc_ref, loff_ref, base_ref, incc_ref,
             cnt_send_sem, cnt_recv_sem, send_sem, recv_sem, local_sem):
        my = lax.axis_index("i")

        bar = pltpu.get_barrier_semaphore()
        for k in range(1, N_DEV):
            pl.semaphore_signal(
                bar, inc=1, device_id=((my + k) % N_DEV,),
                device_id_type=pl.DeviceIdType.MESH,
            )
        pl.semaphore_wait(bar, N_DEV - 1)

        hist = jnp.sum(
            (dest_ref[...] == lax.broadcasted_iota(
                jnp.int32, (dest_ref.shape[0], LANE), 1)).astype(jnp.int32),
            axis=0, keepdims=True,
        )
        cnt_all_ref[pl.ds(my, 1)] = hist[None]
        for k in range(1, N_DEV):
            pltpu.make_async_remote_copy(
                src_ref=cnt_all_ref.at[pl.ds(my, 1)],
                dst_ref=cnt_all_ref.at[pl.ds(my, 1)],
                send_sem=cnt_send_sem, recv_sem=cnt_recv_sem,
                device_id=((my + k) % N_DEV,),
                device_id_type=pl.DeviceIdType.MESH,
            ).start()
        cnt_dummy = pltpu.make_async_remote_copy(
            src_ref=cnt_all_ref.at[pl.ds(0, 1)],
            dst_ref=cnt_all_ref.at[pl.ds(0, 1)],
            send_sem=cnt_send_sem, recv_sem=cnt_recv_sem,
            device_id=(my,), device_id_type=pl.DeviceIdType.MESH,
        )
        for _ in range(N_DEV - 1):
            cnt_dummy.wait_send()
        for _ in range(N_DEV - 1):
            cnt_dummy.wait_recv()

        cp = pltpu.make_async_copy(cnt_all_ref, cnt_sm_ref, local_sem)
        cp.start()
        cp.wait()

        for d in range(N_DEV):
            m = jnp.int32(0)
            lo = jnp.int32(0)
            b = jnp.int32(0)
            for s in range(N_DEV):
                c_sd = cnt_sm_ref[s, 0, d]
                m = m + jnp.where(my == s, c_sd, 0)
                b = b + jnp.where(my > s, c_sd, 0)
            for dp in range(d):
                for s in range(N_DEV):
                    lo = lo + jnp.where(my == s, cnt_sm_ref[s, 0, dp], 0)
            myc_ref[d] = m
            loff_ref[d] = lo
            base_ref[d] = b
        for s in range(N_DEV):
            ic = jnp.int32(0)
            for d in range(N_DEV):
                ic = ic + jnp.where(my == d, cnt_sm_ref[s, 0, d], 0)
            incc_ref[s] = ic

        def chunk_off(c, sz):
            return c & jnp.int32(~(2 * sz - 1))

        c_loc = jnp.int32(0)
        for d in range(N_DEV):
            c_loc = c_loc + jnp.where(my == d, cnt_sm_ref[d, 0, d], 0)
        lo_my = loff_ref[my]
        b_my = base_ref[my]
        for bit in range(MAX_BIT, -1, -1):
            sz = 1 << bit

            @pl.when((c_loc & sz) != 0)
            def _(sz=sz):
                off = chunk_off(c_loc, sz)
                pltpu.make_async_copy(
                    xs_ref.at[pl.ds(lo_my + off, sz)],
                    out_ref.at[pl.ds(b_my + off, sz)],
                    local_sem,
                ).start()

        for k in range(1, N_DEV):
            dd = (my + k) % N_DEV
            c = myc_ref[dd]
            s0 = loff_ref[dd]
            r0 = base_ref[dd]
            for bit in range(MAX_BIT, -1, -1):
                sz = 1 << bit

                @pl.when((c & sz) != 0)
                def _(sz=sz, c=c, s0=s0, r0=r0, dd=dd):
                    off = chunk_off(c, sz)
                    pltpu.make_async_remote_copy(
                        src_ref=xs_ref.at[pl.ds(s0 + off, sz)],
                        dst_ref=out_ref.at[pl.ds(r0 + off, sz)],
                        send_sem=send_sem, recv_sem=recv_sem,
                        device_id=(dd,),
                        device_id_type=pl.DeviceIdType.MESH,
                    ).start()

        def dummy_for(sz):
            return pltpu.make_async_remote_copy(
                src_ref=xs_ref.at[pl.ds(0, sz)],
                dst_ref=out_ref.at[pl.ds(0, sz)],
                send_sem=send_sem, recv_sem=recv_sem,
                device_id=(my,), device_id_type=pl.DeviceIdType.MESH,
            )

        for bit in range(MAX_BIT, -1, -1):
            sz = 1 << bit

            @pl.when((c_loc & sz) != 0)
            def _(sz=sz):
                pltpu.make_async_copy(
                    xs_ref.at[pl.ds(0, sz)],
                    out_ref.at[pl.ds(0, sz)],
                    local_sem,
                ).wait()

        for k in range(1, N_DEV):
            c = myc_ref[(my + k) % N_DEV]
            for bit in range(MAX_BIT, -1, -1):
                sz = 1 << bit

                @pl.when((c & sz) != 0)
                def _(sz=sz):
                    dummy_for(sz).wait_send()

        for k in range(1, N_DEV):
            c = incc_ref[(my + k) % N_DEV]
            for bit in range(MAX_BIT, -1, -1):
                sz = 1 << bit

                @pl.when((c & sz) != 0)
                def _(sz=sz):
                    dummy_for(sz).wait_recv()

    out3 = pl.pallas_call(
        body,
        out_shape=jax.ShapeDtypeStruct((n_rows, SUB, LANE), jnp.bfloat16),
        in_specs=[
            pl.BlockSpec(memory_space=pltpu.VMEM),
            pl.BlockSpec(memory_space=pltpu.VMEM),
        ],
        out_specs=pl.BlockSpec(memory_space=pltpu.VMEM),
        scratch_shapes=[
            pltpu.VMEM((N_DEV, 1, LANE), jnp.int32),
            pltpu.SMEM((N_DEV, 1, LANE), jnp.int32),
            pltpu.SMEM((N_DEV,), jnp.int32),
            pltpu.SMEM((N_DEV,), jnp.int32),
            pltpu.SMEM((N_DEV,), jnp.int32),
            pltpu.SMEM((N_DEV,), jnp.int32),
            pltpu.SemaphoreType.DMA,
            pltpu.SemaphoreType.DMA,
            pltpu.SemaphoreType.DMA,
            pltpu.SemaphoreType.DMA,
            pltpu.SemaphoreType.DMA,
        ],
        compiler_params=pltpu.CompilerParams(collective_id=0),
    )(xs, dest2d)
    return out3.reshape(n_rows, n_cols)


# device time: 24467 ns/iter; 1.7475x vs baseline; 1.0418x over previous
import jax
import jax.numpy as jnp
from jax import lax
from jax.experimental import pallas as pl
from jax.experimental.pallas import tpu as pltpu

N_DEV = 4
SUB = 4
LANE = 128
MAX_BIT = 10


def kernel(x, dest):
    n_rows, n_cols = x.shape
    assert n_cols == SUB * LANE
    dest = dest.astype(jnp.int32)

    key = dest * jnp.int32(2 * n_rows) + jnp.arange(n_rows, dtype=jnp.int32)
    order = jnp.sort(key) & jnp.int32(2 * n_rows - 1)
    xs = x[order].astype(jnp.bfloat16).reshape(n_rows, SUB, LANE)
    dest8 = dest.reshape(n_rows // LANE, LANE)

    def body(xs_ref, dest_ref, out_ref,
             cnt_all_ref, cnt_sm_ref, myc_ref, loff_ref, base_ref, incc_ref,
             cnt_send_sem, cnt_recv_sem, send_sem, recv_sem, local_sem):
        my = lax.axis_index("i")

        bar = pltpu.get_barrier_semaphore()
        for k in range(1, N_DEV):
            pl.semaphore_signal(
                bar, inc=1, device_id=((my + k) % N_DEV,),
                device_id_type=pl.DeviceIdType.MESH,
            )
        pl.semaphore_wait(bar, N_DEV - 1)

        lane = lax.broadcasted_iota(jnp.int32, (1, LANE), 1)
        hist = jnp.zeros((1, LANE), jnp.int32)
        for d in range(N_DEV):
            c_d = jnp.sum(
                (dest_ref[...] == d).astype(jnp.int32), keepdims=True
            ).reshape(1, 1)
            hist = hist + jnp.where(lane == d, c_d, 0)
        cnt_all_ref[pl.ds(my, 1)] = hist[None]
        for k in range(1, N_DEV):
            pltpu.make_async_remote_copy(
                src_ref=cnt_all_ref.at[pl.ds(my, 1)],
                dst_ref=cnt_all_ref.at[pl.ds(my, 1)],
                send_sem=cnt_send_sem, recv_sem=cnt_recv_sem,
                device_id=((my + k) % N_DEV,),
                device_id_type=pl.DeviceIdType.MESH,
            ).start()
        cnt_dummy = pltpu.make_async_remote_copy(
            src_ref=cnt_all_ref.at[pl.ds(0, 1)],
            dst_ref=cnt_all_ref.at[pl.ds(0, 1)],
            send_sem=cnt_send_sem, recv_sem=cnt_recv_sem,
            device_id=(my,), device_id_type=pl.DeviceIdType.MESH,
        )
        for _ in range(N_DEV - 1):
            cnt_dummy.wait_send()
        for _ in range(N_DEV - 1):
            cnt_dummy.wait_recv()

        cp = pltpu.make_async_copy(cnt_all_ref, cnt_sm_ref, local_sem)
        cp.start()
        cp.wait()

        for d in range(N_DEV):
            m = jnp.int32(0)
            lo = jnp.int32(0)
            b = jnp.int32(0)
            for s in range(N_DEV):
                c_sd = cnt_sm_ref[s, 0, d]
                m = m + jnp.where(my == s, c_sd, 0)
                b = b + jnp.where(my > s, c_sd, 0)
            for dp in range(d):
                for s in range(N_DEV):
                    lo = lo + jnp.where(my == s, cnt_sm_ref[s, 0, dp], 0)
            myc_ref[d] = m
            loff_ref[d] = lo
            base_ref[d] = b
        for s in range(N_DEV):
            ic = jnp.int32(0)
            for d in range(N_DEV):
                ic = ic + jnp.where(my == d, cnt_sm_ref[s, 0, d], 0)
            incc_ref[s] = ic

        def chunk_off(c, sz):
            return c & jnp.int32(~(2 * sz - 1))

        c_loc = jnp.int32(0)
        for d in range(N_DEV):
            c_loc = c_loc + jnp.where(my == d, cnt_sm_ref[d, 0, d], 0)
        lo_my = loff_ref[my]
        b_my = base_ref[my]
        for bit in range(MAX_BIT, -1, -1):
            sz = 1 << bit

            @pl.when((c_loc & sz) != 0)
            def _(sz=sz):
                off = chunk_off(c_loc, sz)
                pltpu.make_async_copy(
                    xs_ref.at[pl.ds(lo_my + off, sz)],
                    out_ref.at[pl.ds(b_my + off, sz)],
                    local_sem,
                ).start()

        for k in range(1, N_DEV):
            dd = (my + k) % N_DEV
            c = myc_ref[dd]
            s0 = loff_ref[dd]
            r0 = base_ref[dd]
            for bit in range(MAX_BIT, -1, -1):
                sz = 1 << bit

                @pl.when((c & sz) != 0)
                def _(sz=sz, c=c, s0=s0, r0=r0, dd=dd):
                    off = chunk_off(c, sz)
                    pltpu.make_async_remote_copy(
                        src_ref=xs_ref.at[pl.ds(s0 + off, sz)],
                        dst_ref=out_ref.at[pl.ds(r0 + off, sz)],
                        send_sem=send_sem, recv_sem=recv_sem,
                        device_id=(dd,),
                        device_id_type=pl.DeviceIdType.MESH,
                    ).start()

        def dummy_for(sz):
            return pltpu.make_async_remote_copy(
                src_ref=xs_ref.at[pl.ds(0, sz)],
                dst_ref=out_ref.at[pl.ds(0, sz)],
                send_sem=send_sem, recv_sem=recv_sem,
                device_id=(my,), device_id_type=pl.DeviceIdType.MESH,
            )

        for bit in range(MAX_BIT, -1, -1):
            sz = 1 << bit

            @pl.when((c_loc & sz) != 0)
            def _(sz=sz):
                pltpu.make_async_copy(
                    xs_ref.at[pl.ds(0, sz)],
                    out_ref.at[pl.ds(0, sz)],
                    local_sem,
                ).wait()

        for k in range(1, N_DEV):
            c = myc_ref[(my + k) % N_DEV]
            for bit in range(MAX_BIT, -1, -1):
                sz = 1 << bit

                @pl.when((c & sz) != 0)
                def _(sz=sz):
                    dummy_for(sz).wait_send()

        for k in range(1, N_DEV):
            c = incc_ref[(my + k) % N_DEV]
            for bit in range(MAX_BIT, -1, -1):
                sz = 1 << bit

                @pl.when((c & sz) != 0)
                def _(sz=sz):
                    dummy_for(sz).wait_recv()

    out3 = pl.pallas_call(
        body,
        out_shape=jax.ShapeDtypeStruct((n_rows, SUB, LANE), jnp.bfloat16),
        in_specs=[
            pl.BlockSpec(memory_space=pltpu.VMEM),
            pl.BlockSpec(memory_space=pltpu.VMEM),
        ],
        out_specs=pl.BlockSpec(memory_space=pltpu.VMEM),
        scratch_shapes=[
            pltpu.VMEM((N_DEV, 1, LANE), jnp.int32),
            pltpu.SMEM((N_DEV, 1, LANE), jnp.int32),
            pltpu.SMEM((N_DEV,), jnp.int32),
            pltpu.SMEM((N_DEV,), jnp.int32),
            pltpu.SMEM((N_DEV,), jnp.int32),
            pltpu.SMEM((N_DEV,), jnp.int32),
            pltpu.SemaphoreType.DMA,
            pltpu.SemaphoreType.DMA,
            pltpu.SemaphoreType.DMA,
            pltpu.SemaphoreType.DMA,
            pltpu.SemaphoreType.DMA,
        ],
        compiler_params=pltpu.CompilerParams(collective_id=0),
    )(xs, dest8)
    return out3.reshape(n_rows, n_cols)


# device time: 22225 ns/iter; 1.9238x vs baseline; 1.1009x over previous
import jax
import jax.numpy as jnp
from jax import lax
from jax.experimental import pallas as pl
from jax.experimental.pallas import tpu as pltpu

N_DEV = 4
SUB = 4
LANE = 128
MAX_BIT = 10


def kernel(x, dest):
    n_rows, n_cols = x.shape
    assert n_cols == SUB * LANE
    dest = dest.astype(jnp.int32)

    key = dest * jnp.int32(2 * n_rows) + jnp.arange(n_rows, dtype=jnp.int32)
    order = jnp.sort(key) & jnp.int32(2 * n_rows - 1)
    xs = x[order].astype(jnp.bfloat16).reshape(n_rows, SUB, LANE)
    dest8 = dest.reshape(n_rows // LANE, LANE)

    def body(xs_ref, dest_ref, out_ref,
             stage_ref, cnt_all_ref, cnt_sm_ref, my_sm_ref,
             myc_ref, loff_ref, incc_ref, boff_ref,
             cnt_send_sem, cnt_recv_sem, send_sem, recv_sems, local_sem):
        my = lax.axis_index("i")

        bar = pltpu.get_barrier_semaphore()
        for k in range(1, N_DEV):
            pl.semaphore_signal(
                bar, inc=1, device_id=((my + k) % N_DEV,),
                device_id_type=pl.DeviceIdType.MESH,
            )
        pl.semaphore_wait(bar, N_DEV - 1)

        lane = lax.broadcasted_iota(jnp.int32, (1, LANE), 1)
        hist = jnp.zeros((1, LANE), jnp.int32)
        for d in range(N_DEV):
            c_d = jnp.sum(
                (dest_ref[...] == d).astype(jnp.int32), keepdims=True
            ).reshape(1, 1)
            hist = hist + jnp.where(lane == d, c_d, 0)
        cnt_all_ref[pl.ds(my, 1)] = hist[None]

        for k in range(1, N_DEV):
            pltpu.make_async_remote_copy(
                src_ref=cnt_all_ref.at[pl.ds(my, 1)],
                dst_ref=cnt_all_ref.at[pl.ds(my, 1)],
                send_sem=cnt_send_sem, recv_sem=cnt_recv_sem,
                device_id=((my + k) % N_DEV,),
                device_id_type=pl.DeviceIdType.MESH,
            ).start()

        cp0 = pltpu.make_async_copy(
            cnt_all_ref.at[pl.ds(my, 1)], my_sm_ref, local_sem
        )
        cp0.start()
        cp0.wait()
        lo = jnp.int32(0)
        for d in range(N_DEV):
            c_d = my_sm_ref[0, 0, d]
            myc_ref[d] = c_d
            loff_ref[d] = lo
            lo = lo + c_d

        def chunk_off(c, sz):
            return c & jnp.int32(~(2 * sz - 1))

        for k in range(1, N_DEV):
            dd = (my + k) % N_DEV
            c = myc_ref[dd]
            s0 = loff_ref[dd]
            for bit in range(MAX_BIT, -1, -1):
                sz = 1 << bit

                @pl.when((c & sz) != 0)
                def _(sz=sz, c=c, s0=s0, dd=dd):
                    off = chunk_off(c, sz)
                    pltpu.make_async_remote_copy(
                        src_ref=xs_ref.at[pl.ds(s0 + off, sz)],
                        dst_ref=stage_ref.at[my, pl.ds(off, sz)],
                        send_sem=send_sem, recv_sem=recv_sems.at[my],
                        device_id=(dd,),
                        device_id_type=pl.DeviceIdType.MESH,
                    ).start()

        cnt_dummy = pltpu.make_async_remote_copy(
            src_ref=cnt_all_ref.at[pl.ds(0, 1)],
            dst_ref=cnt_all_ref.at[pl.ds(0, 1)],
            send_sem=cnt_send_sem, recv_sem=cnt_recv_sem,
            device_id=(my,), device_id_type=pl.DeviceIdType.MESH,
        )
        for _ in range(N_DEV - 1):
            cnt_dummy.wait_send()
        for _ in range(N_DEV - 1):
            cnt_dummy.wait_recv()
        cp1 = pltpu.make_async_copy(cnt_all_ref, cnt_sm_ref, local_sem)
        cp1.start()
        cp1.wait()

        bo = jnp.int32(0)
        for s in range(N_DEV):
            ic = jnp.int32(0)
            for d in range(N_DEV):
                ic = ic + jnp.where(my == d, cnt_sm_ref[s, 0, d], 0)
            incc_ref[s] = ic
            boff_ref[s] = bo
            bo = bo + ic

        c_loc = incc_ref[my]
        lo_my = loff_ref[my]
        b_my = boff_ref[my]
        for bit in range(MAX_BIT, -1, -1):
            sz = 1 << bit

            @pl.when((c_loc & sz) != 0)
            def _(sz=sz):
                off = chunk_off(c_loc, sz)
                pltpu.make_async_copy(
                    xs_ref.at[pl.ds(lo_my + off, sz)],
                    out_ref.at[pl.ds(b_my + off, sz)],
                    local_sem,
                ).start()

        def recv_dummy(sz, ss):
            return pltpu.make_async_remote_copy(
                src_ref=xs_ref.at[pl.ds(0, sz)],
                dst_ref=stage_ref.at[0, pl.ds(0, sz)],
                send_sem=send_sem, recv_sem=recv_sems.at[ss],
                device_id=(my,), device_id_type=pl.DeviceIdType.MESH,
            )

        for k in range(1, N_DEV):
            ss = (my + k) % N_DEV
            c = incc_ref[ss]
            b0 = boff_ref[ss]
            for bit in range(MAX_BIT, -1, -1):
                sz = 1 << bit

                @pl.when((c & sz) != 0)
                def _(sz=sz, c=c, ss=ss, b0=b0):
                    recv_dummy(sz, ss).wait_recv()
                    off = chunk_off(c, sz)
                    pltpu.make_async_copy(
                        stage_ref.at[ss, pl.ds(off, sz)],
                        out_ref.at[pl.ds(b0 + off, sz)],
                        local_sem,
                    ).start()

        for s in range(N_DEV):
            c = incc_ref[s]
            for bit in range(MAX_BIT, -1, -1):
                sz = 1 << bit

                @pl.when((c & sz) != 0)
                def _(sz=sz):
                    pltpu.make_async_copy(
                        xs_ref.at[pl.ds(0, sz)],
                        out_ref.at[pl.ds(0, sz)],
                        local_sem,
                    ).wait()

        for k in range(1, N_DEV):
            c = myc_ref[(my + k) % N_DEV]
            for bit in range(MAX_BIT, -1, -1):
                sz = 1 << bit

                @pl.when((c & sz) != 0)
                def _(sz=sz):
                    recv_dummy(sz, 0).wait_send()

    out3 = pl.pallas_call(
        body,
        out_shape=jax.ShapeDtypeStruct((n_rows, SUB, LANE), jnp.bfloat16),
        in_specs=[
            pl.BlockSpec(memory_space=pltpu.VMEM),
            pl.BlockSpec(memory_space=pltpu.VMEM),
        ],
        out_specs=pl.BlockSpec(memory_space=pltpu.VMEM),
        scratch_shapes=[
            pltpu.VMEM((N_DEV, n_rows, SUB, LANE), jnp.bfloat16),
            pltpu.VMEM((N_DEV, 1, LANE), jnp.int32),
            pltpu.SMEM((N_DEV, 1, LANE), jnp.int32),
            pltpu.SMEM((1, 1, LANE), jnp.int32),
            pltpu.SMEM((N_DEV,), jnp.int32),
            pltpu.SMEM((N_DEV,), jnp.int32),
            pltpu.SMEM((N_DEV,), jnp.int32),
            pltpu.SMEM((N_DEV,), jnp.int32),
            pltpu.SemaphoreType.DMA,
            pltpu.SemaphoreType.DMA,
            pltpu.SemaphoreType.DMA,
            pltpu.SemaphoreType.DMA((N_DEV,)),
            pltpu.SemaphoreType.DMA,
        ],
        compiler_params=pltpu.CompilerParams(collective_id=0),
    )(xs, dest8)
    return out3.reshape(n_rows, n_cols)


# device time: 17166 ns/iter; 2.4907x vs baseline; 1.2947x over previous
import jax
import jax.numpy as jnp
from jax import lax
from jax.experimental import pallas as pl
from jax.experimental.pallas import tpu as pltpu

N_DEV = 4
SUB = 4
LANE = 128
MAX_BIT = 10


def kernel(x, dest):
    n_rows, n_cols = x.shape
    assert n_cols == SUB * LANE
    dest = dest.astype(jnp.int32)
    dest8 = dest.reshape(n_rows // LANE, LANE)

    def body(x_ref, dest_ref, out_ref,
             xs_ref, stage_ref, cnt_all_ref, cnt_sm_ref, my_sm_ref,
             myc_ref, loff_ref, incc_ref, boff_ref,
             cnt_send_sem, cnt_recv_sem, send_sem, recv_sems, local_sem):
        my = lax.axis_index("i")

        bar = pltpu.get_barrier_semaphore()
        for k in range(1, N_DEV):
            pl.semaphore_signal(
                bar, inc=1, device_id=((my + k) % N_DEV,),
                device_id_type=pl.DeviceIdType.MESH,
            )
        pl.semaphore_wait(bar, N_DEV - 1)

        lane = lax.broadcasted_iota(jnp.int32, (1, LANE), 1)
        hist = jnp.zeros((1, LANE), jnp.int32)
        for d in range(N_DEV):
            c_d = jnp.sum(
                (dest_ref[...] == d).astype(jnp.int32), keepdims=True
            ).reshape(1, 1)
            hist = hist + jnp.where(lane == d, c_d, 0)
        cnt_all_ref[pl.ds(my, 1)] = hist[None]

        for k in range(1, N_DEV):
            pltpu.make_async_remote_copy(
                src_ref=cnt_all_ref.at[pl.ds(my, 1)],
                dst_ref=cnt_all_ref.at[pl.ds(my, 1)],
                send_sem=cnt_send_sem, recv_sem=cnt_recv_sem,
                device_id=((my + k) % N_DEV,),
                device_id_type=pl.DeviceIdType.MESH,
            ).start()

        cp0 = pltpu.make_async_copy(
            cnt_all_ref.at[pl.ds(my, 1)], my_sm_ref, local_sem
        )
        cp0.start()
        cp0.wait()
        lo = jnp.int32(0)
        for d in range(N_DEV):
            c_d = my_sm_ref[0, 0, d]
            myc_ref[d] = c_d
            loff_ref[d] = lo
            lo = lo + c_d

        d_row = dest_ref[...].reshape(1, n_rows)
        sub4 = lax.broadcasted_iota(jnp.int32, (N_DEV, n_rows), 0)
        oh = (jnp.broadcast_to(d_row, (N_DEV, n_rows)) == sub4)
        oh = oh.astype(jnp.int32)
        lanes_n = lax.broadcasted_iota(jnp.int32, (N_DEV, n_rows), 1)
        c = oh
        sh = 1
        while sh < n_rows:
            rolled = pltpu.roll(c, sh, 1)
            c = c + jnp.where(lanes_n < sh, 0, rolled)
            sh *= 2
        tot = c[:, n_rows - 1:n_rows]
        subs = lax.broadcasted_iota(jnp.int32, (N_DEV, 1), 0)
        loff_v = jnp.zeros((N_DEV, 1), jnp.int32)
        for k in range(1, N_DEV):
            r = pltpu.roll(tot, k, 0)
            loff_v = loff_v + jnp.where(subs < k, 0, r)
        pos = jnp.sum(oh * (c - 1 + loff_v), axis=0, keepdims=True)
        kcol = lax.broadcasted_iota(jnp.int32, (n_rows, 1), 0)
        perm = (jnp.broadcast_to(pos, (n_rows, n_rows)) == kcol)
        xsv = jax.lax.dot(
            perm.astype(jnp.bfloat16),
            x_ref[...].astype(jnp.bfloat16),
            preferred_element_type=jnp.float32,
        )
        xs_ref[...] = xsv.astype(jnp.bfloat16).reshape(n_rows, SUB, LANE)

        def chunk_off(cv, sz):
            return cv & jnp.int32(~(2 * sz - 1))

        for k in range(1, N_DEV):
            dd = (my + k) % N_DEV
            cdd = myc_ref[dd]
            s0 = loff_ref[dd]
            for bit in range(MAX_BIT, -1, -1):
                sz = 1 << bit

                @pl.when((cdd & sz) != 0)
                def _(sz=sz, cdd=cdd, s0=s0, dd=dd):
                    off = chunk_off(cdd, sz)
                    pltpu.make_async_remote_copy(
                        src_ref=xs_ref.at[pl.ds(s0 + off, sz)],
                        dst_ref=stage_ref.at[my, pl.ds(off, sz)],
                        send_sem=send_sem, recv_sem=recv_sems.at[my],
                        device_id=(dd,),
                        device_id_type=pl.DeviceIdType.MESH,
                    ).start()

        cnt_dummy = pltpu.make_async_remote_copy(
            src_ref=cnt_all_ref.at[pl.ds(0, 1)],
            dst_ref=cnt_all_ref.at[pl.ds(0, 1)],
            send_sem=cnt_send_sem, recv_sem=cnt_recv_sem,
            device_id=(my,), device_id_type=pl.DeviceIdType.MESH,
        )
        for _ in range(N_DEV - 1):
            cnt_dummy.wait_send()
        for _ in range(N_DEV - 1):
            cnt_dummy.wait_recv()
        cp1 = pltpu.make_async_copy(cnt_all_ref, cnt_sm_ref, local_sem)
        cp1.start()
        cp1.wait()

        bo = jnp.int32(0)
        for s in range(N_DEV):
            ic = jnp.int32(0)
            for d in range(N_DEV):
                ic = ic + jnp.where(my == d, cnt_sm_ref[s, 0, d], 0)
            incc_ref[s] = ic
            boff_ref[s] = bo
            bo = bo + ic

        c_loc = incc_ref[my]
        lo_my = loff_ref[my]
        b_my = boff_ref[my]
        for bit in range(MAX_BIT, -1, -1):
            sz = 1 << bit

            @pl.when((c_loc & sz) != 0)
            def _(sz=sz):
                off = chunk_off(c_loc, sz)
                pltpu.make_async_copy(
                    xs_ref.at[pl.ds(lo_my + off, sz)],
                    out_ref.at[pl.ds(b_my + off, sz)],
                    local_sem,
                ).start()

        def recv_dummy(sz, ss):
            return pltpu.make_async_remote_copy(
                src_ref=xs_ref.at[pl.ds(0, sz)],
                dst_ref=stage_ref.at[0, pl.ds(0, sz)],
                send_sem=send_sem, recv_sem=recv_sems.at[ss],
                device_id=(my,), device_id_type=pl.DeviceIdType.MESH,
            )

        for k in range(1, N_DEV):
            ss = (my + k) % N_DEV
            cs = incc_ref[ss]
            b0 = boff_ref[ss]
            for bit in range(MAX_BIT, -1, -1):
                sz = 1 << bit

                @pl.when((cs & sz) != 0)
                def _(sz=sz, cs=cs, ss=ss, b0=b0):
                    recv_dummy(sz, ss).wait_recv()
                    off = chunk_off(cs, sz)
                    pltpu.make_async_copy(
                        stage_ref.at[ss, pl.ds(off, sz)],
                        out_ref.at[pl.ds(b0 + off, sz)],
                        local_sem,
                    ).start()

        for s in range(N_DEV):
            cs = incc_ref[s]
            for bit in range(MAX_BIT, -1, -1):
                sz = 1 << bit

                @pl.when((cs & sz) != 0)
                def _(sz=sz):
                    pltpu.make_async_copy(
                        xs_ref.at[pl.ds(0, sz)],
                        out_ref.at[pl.ds(0, sz)],
                        local_sem,
                    ).wait()

        for k in range(1, N_DEV):
            cdd = myc_ref[(my + k) % N_DEV]
            for bit in range(MAX_BIT, -1, -1):
                sz = 1 << bit

                @pl.when((cdd & sz) != 0)
                def _(sz=sz):
                    recv_dummy(sz, 0).wait_send()

    out3 = pl.pallas_call(
        body,
        out_shape=jax.ShapeDtypeStruct((n_rows, SUB, LANE), jnp.bfloat16),
        in_specs=[
            pl.BlockSpec(memory_space=pltpu.VMEM),
            pl.BlockSpec(memory_space=pltpu.VMEM),
        ],
        out_specs=pl.BlockSpec(memory_space=pltpu.VMEM),
        scratch_shapes=[
            pltpu.VMEM((n_rows, SUB, LANE), jnp.bfloat16),
            pltpu.VMEM((N_DEV, n_rows, SUB, LANE), jnp.bfloat16),
            pltpu.VMEM((N_DEV, 1, LANE), jnp.int32),
            pltpu.SMEM((N_DEV, 1, LANE), jnp.int32),
            pltpu.SMEM((1, 1, LANE), jnp.int32),
            pltpu.SMEM((N_DEV,), jnp.int32),
            pltpu.SMEM((N_DEV,), jnp.int32),
            pltpu.SMEM((N_DEV,), jnp.int32),
            pltpu.SMEM((N_DEV,), jnp.int32),
            pltpu.SemaphoreType.DMA,
            pltpu.SemaphoreType.DMA,
            pltpu.SemaphoreType.DMA,
            pltpu.SemaphoreType.DMA((N_DEV,)),
            pltpu.SemaphoreType.DMA,
        ],
        compiler_params=pltpu.CompilerParams(collective_id=0),
    )(x, dest8)
    return out3.reshape(n_rows, n_cols)
